# Initial kernel scaffold; baseline (speedup 1.0000x reference)
import jax
import jax.numpy as jnp
from jax.experimental import pallas as pl


def kernel(x, y, labels, memory_da, memory, params):
    raise NotImplementedError("write your pallas kernel here")


# speedup vs baseline: 1.1497x; 1.1497x over previous
import functools

import jax
import jax.numpy as jnp
from jax.experimental import pallas as pl
from jax.experimental.pallas import tpu as pltpu

B = 512
D = 32
M = 16384
TM = 2048
NT = M // TM


def _rowsum_body(params_ref, x_ref, mda_ref, z_ref):
    i = pl.program_id(0)
    inv_t = 1.0 / params_ref[1]
    x = x_ref[...] * inv_t
    mda = mda_ref[...]
    s = jax.lax.dot_general(
        x, mda, (((1,), (1,)), ((), ())), preferred_element_type=jnp.float32
    )
    part = jnp.sum(jnp.exp(s), axis=1, keepdims=True)

    @pl.when(i == 0)
    def _():
        z_ref[...] = part

    @pl.when(i > 0)
    def _():
        z_ref[...] += part


def _normalize_body(params_ref, x_ref, mda_ref, z_ref, o_ref):
    inv_t = 1.0 / params_ref[1]
    x = x_ref[...] * inv_t
    mda = mda_ref[...]
    s = jax.lax.dot_general(
        x, mda, (((1,), (1,)), ((), ())), preferred_element_type=jnp.float32
    )
    o_ref[...] = jnp.exp(s) / z_ref[...]


@functools.partial(jax.jit, static_argnames=())
def _nce_forward(x, mda, params):
    z = pl.pallas_call(
        _rowsum_body,
        grid=(NT,),
        in_specs=[
            pl.BlockSpec(memory_space=pltpu.SMEM),
            pl.BlockSpec((B, D), lambda i: (0, 0)),
            pl.BlockSpec((TM, D), lambda i: (i, 0)),
        ],
        out_specs=pl.BlockSpec((B, 1), lambda i: (0, 0)),
        out_shape=jax.ShapeDtypeStruct((B, 1), jnp.float32),
    )(params, x, mda)

    out = pl.pallas_call(
        _normalize_body,
        grid=(NT,),
        in_specs=[
            pl.BlockSpec(memory_space=pltpu.SMEM),
            pl.BlockSpec((B, D), lambda i: (0, 0)),
            pl.BlockSpec((TM, D), lambda i: (i, 0)),
            pl.BlockSpec((B, 1), lambda i: (0, 0)),
        ],
        out_specs=pl.BlockSpec((B, TM), lambda i: (0, i)),
        out_shape=jax.ShapeDtypeStruct((B, M), jnp.float32),
    )(params, x, mda, z)
    return out


def kernel(x, y, labels, memory_da, memory, params):
    mda = memory_da[:, 1:]
    return _nce_forward(x, mda, params)


# speedup vs baseline: 1.3904x; 1.2093x over previous
import functools

import jax
import jax.numpy as jnp
from jax.experimental import pallas as pl
from jax.experimental.pallas import tpu as pltpu

B = 512
D = 32
M = 16384
TM = 2048
NT = M // TM


def _nce_body(params_ref, x_ref, mda_ref, o_ref, e_ref, z_ref):
    p = pl.program_id(0)
    j = pl.program_id(1)

    @pl.when(p == 0)
    def _():
        inv_t = 1.0 / params_ref[1]
        x = x_ref[...] * inv_t
        mda = mda_ref[...]
        s = jax.lax.dot_general(
            x, mda, (((1,), (1,)), ((), ())), preferred_element_type=jnp.float32
        )
        e = jnp.exp(s)
        e_ref[j] = e
        part = jnp.sum(e, axis=1, keepdims=True)

        @pl.when(j == 0)
        def _():
            z_ref[...] = part

        @pl.when(j > 0)
        def _():
            z_ref[...] += part

        @pl.when(j == NT - 1)
        def _():
            z_ref[...] = 1.0 / z_ref[...]

    @pl.when(p == 1)
    def _():
        o_ref[...] = e_ref[j] * z_ref[...]


@functools.partial(jax.jit, static_argnames=())
def _nce_forward(x, mda, params):
    return pl.pallas_call(
        _nce_body,
        grid=(2, NT),
        in_specs=[
            pl.BlockSpec(memory_space=pltpu.SMEM),
            pl.BlockSpec((B, D), lambda p, j: (0, 0)),
            pl.BlockSpec((TM, D), lambda p, j: (j * (1 - p), 0)),
        ],
        out_specs=pl.BlockSpec((B, TM), lambda p, j: (0, j * p)),
        out_shape=jax.ShapeDtypeStruct((B, M), jnp.float32),
        scratch_shapes=[
            pltpu.VMEM((NT, B, TM), jnp.float32),
            pltpu.VMEM((B, 1), jnp.float32),
        ],
    )(params, x, mda)


def kernel(x, y, labels, memory_da, memory, params):
    mda = memory_da[:, 1:]
    return _nce_forward(x, mda, params)
